# baseline (speedup 1.0000x reference)
import functools

import jax
import jax.numpy as jnp
from jax import lax
from jax.experimental import pallas as pl
from jax.experimental.pallas import tpu as pltpu
from jax.experimental.pallas import tpu_sc as plsc

NC = 2
NS = 16
NW = NC * NS
CH = 128


def _emb_kernel(n_chunks, d):
    mesh = plsc.VectorSubcoreMesh(core_axis_name="c", subcore_axis_name="s")

    @functools.partial(
        pl.kernel,
        mesh=mesh,
        compiler_params=pltpu.CompilerParams(use_tc_tiling_on_sc=False),
        out_type=jax.ShapeDtypeStruct((NW, n_chunks, CH, d), jnp.float32),
        scratch_types=[
            pltpu.VMEM((n_chunks, CH), jnp.int32),
            pltpu.VMEM((CH, d), jnp.float32),
            pltpu.SemaphoreType.DMA,
        ],
    )
    def k(x_hbm, table_hbm, out_hbm, idx_v, rows_v, sem):
        wid = lax.axis_index("s") * NC + lax.axis_index("c")
        pltpu.sync_copy(x_hbm.at[wid], idx_v)

        def body(g, carry):
            pltpu.async_copy(table_hbm.at[idx_v.at[g]], rows_v, sem).wait()
            pltpu.sync_copy(rows_v, out_hbm.at[wid, g])
            return carry

        lax.fori_loop(0, n_chunks, body, 0)

    return k


def kernel(x, table):
    b, h = x.shape
    v, d = table.shape
    n = b * h
    assert n % (NW * CH) == 0
    n_chunks = n // (NW * CH)
    xr = jnp.asarray(x, jnp.int32).reshape(NW, n_chunks, CH)
    out = _emb_kernel(n_chunks, d)(xr, table)
    return out.reshape(b, h, d)


# speedup vs baseline: 1.0456x; 1.0456x over previous
import functools

import jax
import jax.numpy as jnp
from jax import lax
from jax.experimental import pallas as pl
from jax.experimental.pallas import tpu as pltpu
from jax.experimental.pallas import tpu_sc as plsc

NC = 2
NS = 16
NW = NC * NS
CH = 128
KG = 5
SC_ROWS = CH * KG


def _emb_kernel(n_super, d):
    mesh = plsc.VectorSubcoreMesh(core_axis_name="c", subcore_axis_name="s")

    @functools.partial(
        pl.kernel,
        mesh=mesh,
        compiler_params=pltpu.CompilerParams(use_tc_tiling_on_sc=False),
        out_type=jax.ShapeDtypeStruct((NW, n_super, SC_ROWS, d), jnp.float32),
        scratch_types=[
            pltpu.VMEM((n_super * KG, CH), jnp.int32),
            pltpu.VMEM((2, SC_ROWS, d), jnp.float32),
            pltpu.SemaphoreType.DMA,
            pltpu.SemaphoreType.DMA,
        ],
    )
    def k(x_hbm, table_hbm, out_hbm, idx_v, rows_v, sem0, sem1):
        wid = lax.axis_index("s") * NC + lax.axis_index("c")
        pltpu.sync_copy(x_hbm.at[wid], idx_v)
        sems = (sem0, sem1)

        def fire(s, b):
            for j in range(KG):
                pltpu.async_copy(
                    table_hbm.at[idx_v.at[s * KG + j]],
                    rows_v.at[b, pl.ds(j * CH, CH)],
                    sems[b],
                )

        def drain_store(s, b):
            for j in range(KG):
                pltpu.make_async_copy(
                    table_hbm.at[idx_v.at[s * KG + j]],
                    rows_v.at[b, pl.ds(j * CH, CH)],
                    sems[b],
                ).wait()
            pltpu.sync_copy(rows_v.at[b], out_hbm.at[wid, s])

        fire(0, 0)
        fire(1, 1)

        def body(i, carry):
            s0 = 2 * i
            drain_store(s0, 0)
            fire(s0 + 2, 0)
            drain_store(s0 + 1, 1)
            fire(s0 + 3, 1)
            return carry

        lax.fori_loop(0, n_super // 2 - 1, body, 0)
        drain_store(n_super - 2, 0)
        drain_store(n_super - 1, 1)

    return k


def kernel(x, table):
    b, h = x.shape
    v, d = table.shape
    n = b * h
    assert n % (NW * SC_ROWS) == 0
    n_super = n // (NW * SC_ROWS)
    assert n_super % 2 == 0
    xr = jnp.asarray(x, jnp.int32).reshape(NW, n_super * KG, CH)
    out = _emb_kernel(n_super, d)(xr, table)
    return out.reshape(b, h, d)
